# Initial kernel scaffold; baseline (speedup 1.0000x reference)
import jax
import jax.numpy as jnp
from jax.experimental import pallas as pl


def kernel(compressed_matrix):
    raise NotImplementedError("write your pallas kernel here")


# speedup vs baseline: 64.6745x; 64.6745x over previous
import functools

import jax
import jax.numpy as jnp
from jax import lax
from jax.experimental import pallas as pl
from jax.experimental.pallas import tpu as pltpu
from jax.experimental.pallas import tpu_sc as plsc

N = 4096
L = N * (N - 1) // 2
NUM_TILES = 32
ROWS_PER_TILE = N // NUM_TILES
LANES = 16
CHUNKS = N // LANES
WIN = N + LANES
PAD_LEN = 1 + L + 31


def _body(padded_hbm, out_hbm, win_v, row_v):
    cid = lax.axis_index("c")
    sid = lax.axis_index("s")
    wid = sid * 2 + cid

    lane = lax.iota(jnp.int32, LANES)

    def zero_chunk(c, _):
        row_v[pl.ds(c * LANES, LANES)] = jnp.zeros((LANES,), jnp.float32)
        return 0
    lax.fori_loop(0, CHUNKS, zero_chunk, 0)

    def do_row(t, _):
        i = wid + (ROWS_PER_TILE - 1 - t) * NUM_TILES
        s = i * (N - 2) - (i * (i - 1)) // 2
        r = pl.multiple_of((s // LANES) * LANES, LANES)
        d = s - r
        cb = (i + 1) // LANES

        pltpu.sync_copy(padded_hbm.at[pl.ds(r, WIN)], win_v)

        def do_chunk(c, _):
            idx = lane + (d + c * LANES)
            v = plsc.load_gather(win_v, [idx])
            keep = (c * LANES + lane) >= (i + 1)
            row_v[pl.ds(c * LANES, LANES)] = jnp.where(keep, v, 0.0)
            return 0
        lax.fori_loop(cb, CHUNKS, do_chunk, 0)

        pltpu.sync_copy(row_v, out_hbm.at[pl.ds(i * N, N)])
        return 0

    lax.fori_loop(0, ROWS_PER_TILE, do_row, 0)


@jax.jit
def kernel(compressed_matrix):
    padded = jnp.concatenate([
        jnp.zeros((1,), jnp.float32),
        compressed_matrix,
        jnp.zeros((PAD_LEN - 1 - L,), jnp.float32),
    ])
    run = pl.kernel(
        _body,
        out_type=jax.ShapeDtypeStruct((N * N,), jnp.float32),
        mesh=plsc.VectorSubcoreMesh(core_axis_name="c", subcore_axis_name="s"),
        scratch_types=[
            pltpu.VMEM((WIN,), jnp.float32),
            pltpu.VMEM((N,), jnp.float32),
        ],
        compiler_params=pltpu.CompilerParams(needs_layout_passes=False),
    )
    return run(padded).reshape(N, N)


# speedup vs baseline: 173.2286x; 2.6785x over previous
import jax
import jax.numpy as jnp
from jax import lax
from jax.experimental import pallas as pl
from jax.experimental.pallas import tpu as pltpu
from jax.experimental.pallas import tpu_sc as plsc

N = 4096
L = N * (N - 1) // 2
NUM_TILES = 32
ROWS_PER_TILE = N // NUM_TILES
LANES = 16
CHUNKS = N // LANES
WIN = N + LANES
UNROLL = 8
GROUPS = CHUNKS // UNROLL


def _row_id(wid, t):
    return wid + (ROWS_PER_TILE - 1 - t) * NUM_TILES


def _win_start(i):
    s = i * (N - 2) - (i * (i - 1)) // 2 - 1
    r = (s // LANES) * LANES
    r = pl.multiple_of(jnp.clip(r, 0, L - WIN), LANES)
    return s, r


def _body(comp_hbm, out_hbm, win0, win1, row0, row1, si0, si1, so0, so1):
    cid = lax.axis_index("c")
    sid = lax.axis_index("s")
    wid = sid * 2 + cid

    lane = lax.iota(jnp.int32, LANES)

    def zero_chunk(c, _):
        z = jnp.zeros((LANES,), jnp.float32)
        row0[pl.ds(c * LANES, LANES)] = z
        row1[pl.ds(c * LANES, LANES)] = z
        return 0
    lax.fori_loop(0, CHUNKS, zero_chunk, 0)

    def start_in(t, win, sem):
        i = _row_id(wid, t)
        _, r = _win_start(i)
        pltpu.make_async_copy(comp_hbm.at[pl.ds(r, WIN)], win, sem).start()

    def wait_in(win, sem):
        pltpu.make_async_copy(comp_hbm.at[pl.ds(0, WIN)], win, sem).wait()

    def wait_out(row, sem):
        pltpu.make_async_copy(row, out_hbm.at[pl.ds(0, N)], sem).wait()

    def build(rowbuf, win, i):
        s, r = _win_start(i)
        d = s - r
        cb = (i + 1) // LANES
        ip1 = i + 1

        def grp(g, _):
            base = g * (UNROLL * LANES)
            for k in range(UNROLL):
                c16 = base + k * LANES
                idx = jnp.maximum(lane + (d + c16), 0)
                v = plsc.load_gather(win, [idx])
                keep = (lane + c16) >= ip1
                rowbuf[pl.ds(c16, LANES)] = jnp.where(keep, v, 0.0)
            return 0
        lax.fori_loop(cb // UNROLL, GROUPS, grp, 0)

    def half(t, win, row, sem_in, sem_out, win_n, sem_in_n, first, last):
        i = _row_id(wid, t)
        wait_in(win, sem_in)
        if last is None:
            start_in(t + 1, win_n, sem_in_n)
        else:
            pl.when(jnp.logical_not(last))(
                lambda: start_in(t + 1, win_n, sem_in_n))
        pl.when(jnp.logical_not(first))(lambda: wait_out(row, sem_out))
        build(row, win, i)
        pltpu.make_async_copy(
            row, out_hbm.at[pl.ds(i * N, N)], sem_out).start()

    start_in(0, win0, si0)

    def pair(g, _):
        t0 = 2 * g
        half(t0, win0, row0, si0, so0, win1, si1, g == 0, None)
        half(t0 + 1, win1, row1, si1, so1, win0, si0, g == 0,
             g == ROWS_PER_TILE // 2 - 1)
        return 0
    lax.fori_loop(0, ROWS_PER_TILE // 2, pair, 0)

    wait_out(row0, so0)
    wait_out(row1, so1)


@jax.jit
def kernel(compressed_matrix):
    run = pl.kernel(
        _body,
        out_type=jax.ShapeDtypeStruct((N * N,), jnp.float32),
        mesh=plsc.VectorSubcoreMesh(core_axis_name="c", subcore_axis_name="s"),
        scratch_types=[
            pltpu.VMEM((WIN,), jnp.float32),
            pltpu.VMEM((WIN,), jnp.float32),
            pltpu.VMEM((N,), jnp.float32),
            pltpu.VMEM((N,), jnp.float32),
            pltpu.SemaphoreType.DMA,
            pltpu.SemaphoreType.DMA,
            pltpu.SemaphoreType.DMA,
            pltpu.SemaphoreType.DMA,
        ],
        compiler_params=pltpu.CompilerParams(needs_layout_passes=False),
    )
    return run(compressed_matrix).reshape(N, N)


# speedup vs baseline: 250.5225x; 1.4462x over previous
import jax
import jax.numpy as jnp
from jax import lax
from jax.experimental import pallas as pl
from jax.experimental.pallas import tpu as pltpu
from jax.experimental.pallas import tpu_sc as plsc

N = 4096
L = N * (N - 1) // 2
NUM_TILES = 32
ROWS_PER_TILE = N // NUM_TILES
LANES = 16
CHUNKS = N // LANES
WIN = N + LANES
UNROLL = 8
GROUPS = CHUNKS // UNROLL


def _row_id(wid, t):
    return wid + (ROWS_PER_TILE - 1 - t) * NUM_TILES


def _win_start(i):
    s = i * (N - 2) - (i * (i - 1)) // 2 - 1
    r = (s // LANES) * LANES
    r = pl.multiple_of(jnp.clip(r, 0, L - WIN), LANES)
    return s, r


def _body(comp_hbm, out_hbm, win0, win1, row0, row1, si0, si1, so0, so1):
    cid = lax.axis_index("c")
    sid = lax.axis_index("s")
    wid = sid * 2 + cid

    lane = lax.iota(jnp.int32, LANES)

    def zero_chunk(c, _):
        z = jnp.zeros((LANES,), jnp.float32)
        row0[pl.ds(c * LANES, LANES)] = z
        row1[pl.ds(c * LANES, LANES)] = z
        return 0
    lax.fori_loop(0, CHUNKS, zero_chunk, 0)

    def start_in(t, win, sem):
        i = _row_id(wid, t)
        _, r = _win_start(i)
        pltpu.make_async_copy(comp_hbm.at[pl.ds(r, WIN)], win, sem).start()

    def wait_in(win, sem):
        pltpu.make_async_copy(comp_hbm.at[pl.ds(0, WIN)], win, sem).wait()

    def wait_out(row, sem):
        pltpu.make_async_copy(row, out_hbm.at[0], sem).wait()

    def build(rowbuf, win, i):
        s, r = _win_start(i)
        d = s - r
        cb = (i + 1) // LANES
        ip1 = i + 1

        def grp(g, _):
            base = g * (UNROLL * LANES)
            for k in range(UNROLL):
                c16 = base + k * LANES
                idx = jnp.maximum(lane + (d + c16), 0)
                v = plsc.load_gather(win, [idx])
                keep = (lane + c16) >= ip1
                rowbuf[pl.ds(c16, LANES)] = jnp.where(keep, v, 0.0)
            return 0
        lax.fori_loop(cb // UNROLL, GROUPS, grp, 0)

    def half(t, win, row, sem_in, sem_out, win_n, sem_in_n, first, last):
        i = _row_id(wid, t)
        wait_in(win, sem_in)
        if last is None:
            start_in(t + 1, win_n, sem_in_n)
        else:
            pl.when(jnp.logical_not(last))(
                lambda: start_in(t + 1, win_n, sem_in_n))
        pl.when(jnp.logical_not(first))(lambda: wait_out(row, sem_out))
        build(row, win, i)
        pltpu.make_async_copy(row, out_hbm.at[i], sem_out).start()

    start_in(0, win0, si0)

    def pair(g, _):
        t0 = 2 * g
        half(t0, win0, row0, si0, so0, win1, si1, g == 0, None)
        half(t0 + 1, win1, row1, si1, so1, win0, si0, g == 0,
             g == ROWS_PER_TILE // 2 - 1)
        return 0
    lax.fori_loop(0, ROWS_PER_TILE // 2, pair, 0)

    wait_out(row0, so0)
    wait_out(row1, so1)


@jax.jit
def kernel(compressed_matrix):
    run = pl.kernel(
        _body,
        out_type=jax.ShapeDtypeStruct((N, N), jnp.float32),
        mesh=plsc.VectorSubcoreMesh(core_axis_name="c", subcore_axis_name="s"),
        scratch_types=[
            pltpu.VMEM((WIN,), jnp.float32),
            pltpu.VMEM((WIN,), jnp.float32),
            pltpu.VMEM((N,), jnp.float32),
            pltpu.VMEM((N,), jnp.float32),
            pltpu.SemaphoreType.DMA,
            pltpu.SemaphoreType.DMA,
            pltpu.SemaphoreType.DMA,
            pltpu.SemaphoreType.DMA,
        ],
        compiler_params=pltpu.CompilerParams(needs_layout_passes=False),
    )
    return run(compressed_matrix)


# speedup vs baseline: 259.7349x; 1.0368x over previous
import jax
import jax.numpy as jnp
from jax import lax
from jax.experimental import pallas as pl
from jax.experimental.pallas import tpu as pltpu
from jax.experimental.pallas import tpu_sc as plsc

N = 4096
L = N * (N - 1) // 2
NUM_TILES = 32
ROWS_PER_TILE = N // NUM_TILES
LANES = 16
CHUNKS = N // LANES
WIN = N + LANES
UNROLL = 8
GROUPS = CHUNKS // UNROLL


def _row_id(wid, t):
    return wid + (ROWS_PER_TILE - 1 - t) * NUM_TILES


def _win_start(i):
    s = i * (N - 2) - (i * (i - 1)) // 2 - 1
    r = (s // LANES) * LANES
    r = pl.multiple_of(jnp.clip(r, 0, L - WIN), LANES)
    return s, r


def _body(comp_hbm, out_hbm, win0, win1, row0, row1, si0, si1, so0, so1):
    cid = lax.axis_index("c")
    sid = lax.axis_index("s")
    wid = sid * 2 + cid

    lane = lax.iota(jnp.int32, LANES)

    def zero_chunk(c, _):
        z = jnp.zeros((LANES,), jnp.float32)
        row0[pl.ds(c * LANES, LANES)] = z
        row1[pl.ds(c * LANES, LANES)] = z
        return 0
    lax.fori_loop(0, CHUNKS, zero_chunk, 0)

    def start_in(t, win, sem):
        i = _row_id(wid, t)
        _, r = _win_start(i)
        pltpu.make_async_copy(comp_hbm.at[pl.ds(r, WIN)], win, sem).start()

    def wait_in(win, sem):
        pltpu.make_async_copy(comp_hbm.at[pl.ds(0, WIN)], win, sem).wait()

    def wait_out(row, sem):
        pltpu.make_async_copy(row, out_hbm.at[0], sem).wait()

    def build(rowbuf, win, i):
        s, r = _win_start(i)
        d = s - r
        cb = (i + 1) // LANES

        @pl.when(i < N - 1)
        def _():
            c16 = cb * LANES
            idx = jnp.maximum(lane + (d + c16), 0)
            v = plsc.load_gather(win, [idx])
            keep = (lane + c16) >= i + 1
            rowbuf[pl.ds(c16, LANES)] = jnp.where(keep, v, 0.0)

        @plsc.parallel_loop(cb + 1, CHUNKS, unroll=UNROLL)
        def _(c):
            c16 = c * LANES
            v = plsc.load_gather(win, [lane + (d + c16)])
            rowbuf[pl.ds(c16, LANES)] = v

    def half(t, win, row, sem_in, sem_out, win_n, sem_in_n, first, last):
        i = _row_id(wid, t)
        wait_in(win, sem_in)
        if last is None:
            start_in(t + 1, win_n, sem_in_n)
        else:
            pl.when(jnp.logical_not(last))(
                lambda: start_in(t + 1, win_n, sem_in_n))
        pl.when(jnp.logical_not(first))(lambda: wait_out(row, sem_out))
        build(row, win, i)
        pltpu.make_async_copy(row, out_hbm.at[i], sem_out).start()

    start_in(0, win0, si0)

    def pair(g, _):
        t0 = 2 * g
        half(t0, win0, row0, si0, so0, win1, si1, g == 0, None)
        half(t0 + 1, win1, row1, si1, so1, win0, si0, g == 0,
             g == ROWS_PER_TILE // 2 - 1)
        return 0
    lax.fori_loop(0, ROWS_PER_TILE // 2, pair, 0)

    wait_out(row0, so0)
    wait_out(row1, so1)


@jax.jit
def kernel(compressed_matrix):
    run = pl.kernel(
        _body,
        out_type=jax.ShapeDtypeStruct((N, N), jnp.float32),
        mesh=plsc.VectorSubcoreMesh(core_axis_name="c", subcore_axis_name="s"),
        scratch_types=[
            pltpu.VMEM((WIN,), jnp.float32),
            pltpu.VMEM((WIN,), jnp.float32),
            pltpu.VMEM((N,), jnp.float32),
            pltpu.VMEM((N,), jnp.float32),
            pltpu.SemaphoreType.DMA,
            pltpu.SemaphoreType.DMA,
            pltpu.SemaphoreType.DMA,
            pltpu.SemaphoreType.DMA,
        ],
        compiler_params=pltpu.CompilerParams(needs_layout_passes=False),
    )
    return run(compressed_matrix)


# speedup vs baseline: 297.0730x; 1.1438x over previous
import jax
import jax.numpy as jnp
from jax import lax
from jax.experimental import pallas as pl
from jax.experimental.pallas import tpu as pltpu
from jax.experimental.pallas import tpu_sc as plsc

N = 4096
L = N * (N - 1) // 2
NUM_TILES = 32
ROWS_PER_TILE = N // NUM_TILES
LANES = 16
CHUNKS = N // LANES
WIN = N + LANES
UNROLL = 8
GROUPS = CHUNKS // UNROLL


CLASS_STEP = 32
NUM_CLS = CHUNKS // CLASS_STEP + 1


def _row_id(wid, t):
    return wid + (ROWS_PER_TILE - 1 - t) * NUM_TILES


def _win_params(i):
    s = i * (N - 2) - (i * (i - 1)) // 2 - 1
    cb = (i + 1) // LANES
    k = cb // CLASS_STEP
    base = k * (CLASS_STEP * LANES)
    wlen = WIN - base
    r = ((s + base) // LANES) * LANES
    r = pl.multiple_of(jnp.clip(r, 0, L - wlen), LANES)
    return s - r, r, cb, k


def _body(comp_hbm, out_hbm, win0, win1, row0, row1, si0, si1, so0, so1):
    cid = lax.axis_index("c")
    sid = lax.axis_index("s")
    wid = sid * 2 + cid

    lane = lax.iota(jnp.int32, LANES)

    def zero_chunk(c, _):
        z = jnp.zeros((LANES,), jnp.float32)
        row0[pl.ds(c * LANES, LANES)] = z
        row1[pl.ds(c * LANES, LANES)] = z
        return 0
    lax.fori_loop(0, CHUNKS, zero_chunk, 0)

    def start_in(t, win, sem):
        i = _row_id(wid, t)
        _, r, _, k = _win_params(i)
        for ks in range(NUM_CLS):
            wlen = WIN - ks * (CLASS_STEP * LANES)
            pl.when(k == ks)(lambda wlen=wlen: pltpu.make_async_copy(
                comp_hbm.at[pl.ds(r, wlen)], win.at[pl.ds(0, wlen)],
                sem).start())

    def wait_in(t, win, sem):
        i = _row_id(wid, t)
        _, _, _, k = _win_params(i)
        for ks in range(NUM_CLS):
            wlen = WIN - ks * (CLASS_STEP * LANES)
            pl.when(k == ks)(lambda wlen=wlen: pltpu.make_async_copy(
                comp_hbm.at[pl.ds(0, wlen)], win.at[pl.ds(0, wlen)],
                sem).wait())

    def wait_out(row, sem):
        pltpu.make_async_copy(row, out_hbm.at[0], sem).wait()

    def build(rowbuf, win, i):
        d, _, cb, _ = _win_params(i)

        @pl.when(i < N - 1)
        def _():
            c16 = cb * LANES
            idx = jnp.maximum(lane + (d + c16), 0)
            v = plsc.load_gather(win, [idx])
            keep = (lane + c16) >= i + 1
            rowbuf[pl.ds(c16, LANES)] = jnp.where(keep, v, 0.0)

        @plsc.parallel_loop(cb + 1, CHUNKS, unroll=UNROLL)
        def _(c):
            c16 = c * LANES
            v = plsc.load_gather(win, [lane + (d + c16)])
            rowbuf[pl.ds(c16, LANES)] = v

    def half(t, win, row, sem_in, sem_out, win_n, sem_in_n, first, last):
        i = _row_id(wid, t)
        wait_in(t, win, sem_in)
        if last is None:
            start_in(t + 1, win_n, sem_in_n)
        else:
            pl.when(jnp.logical_not(last))(
                lambda: start_in(t + 1, win_n, sem_in_n))
        pl.when(jnp.logical_not(first))(lambda: wait_out(row, sem_out))
        build(row, win, i)
        pltpu.make_async_copy(row, out_hbm.at[i], sem_out).start()

    start_in(0, win0, si0)

    def pair(g, _):
        t0 = 2 * g
        half(t0, win0, row0, si0, so0, win1, si1, g == 0, None)
        half(t0 + 1, win1, row1, si1, so1, win0, si0, g == 0,
             g == ROWS_PER_TILE // 2 - 1)
        return 0
    lax.fori_loop(0, ROWS_PER_TILE // 2, pair, 0)

    wait_out(row0, so0)
    wait_out(row1, so1)


@jax.jit
def kernel(compressed_matrix):
    run = pl.kernel(
        _body,
        out_type=jax.ShapeDtypeStruct((N, N), jnp.float32),
        mesh=plsc.VectorSubcoreMesh(core_axis_name="c", subcore_axis_name="s"),
        scratch_types=[
            pltpu.VMEM((WIN,), jnp.float32),
            pltpu.VMEM((WIN,), jnp.float32),
            pltpu.VMEM((N,), jnp.float32),
            pltpu.VMEM((N,), jnp.float32),
            pltpu.SemaphoreType.DMA,
            pltpu.SemaphoreType.DMA,
            pltpu.SemaphoreType.DMA,
            pltpu.SemaphoreType.DMA,
        ],
        compiler_params=pltpu.CompilerParams(needs_layout_passes=False),
    )
    return run(compressed_matrix)


# speedup vs baseline: 540.4644x; 1.8193x over previous
import jax
import jax.numpy as jnp
from jax import lax
from jax.experimental import pallas as pl
from jax.experimental.pallas import tpu as pltpu
from jax.experimental.pallas import tpu_sc as plsc

N = 4096
L = N * (N - 1) // 2
NUM_TILES = 32
ROWS_PER_TILE = N // NUM_TILES
LANES = 16
CHUNKS = N // LANES
WIN = N + LANES
UNROLL = 8
DEPTH = 4
CLASS_STEP = 32
NUM_CLS = CHUNKS // CLASS_STEP + 1


def _row_id(wid, t):
    return wid + (ROWS_PER_TILE - 1 - t) * NUM_TILES


def _win_params(i):
    s = i * (N - 2) - (i * (i - 1)) // 2 - 1
    cb = (i + 1) // LANES
    k = cb // CLASS_STEP
    base = k * (CLASS_STEP * LANES)
    wlen = WIN - base
    r = ((s + base) // LANES) * LANES
    r = pl.multiple_of(jnp.clip(r, 0, L - wlen), LANES)
    return s - r, r, cb, k


def _body(comp_hbm, out_hbm, *scratch):
    wins = scratch[0:DEPTH]
    rows = scratch[DEPTH:2 * DEPTH]
    sin = scratch[2 * DEPTH:3 * DEPTH]
    sout = scratch[3 * DEPTH:4 * DEPTH]

    cid = lax.axis_index("c")
    sid = lax.axis_index("s")
    wid = sid * 2 + cid

    lane = lax.iota(jnp.int32, LANES)

    def zero_chunk(c, _):
        z = jnp.zeros((LANES,), jnp.float32)
        for rb in rows:
            rb[pl.ds(c * LANES, LANES)] = z
        return 0
    lax.fori_loop(0, CHUNKS, zero_chunk, 0)

    def start_in(t, win, sem):
        i = _row_id(wid, t)
        _, r, _, k = _win_params(i)
        for ks in range(NUM_CLS):
            wlen = WIN - ks * (CLASS_STEP * LANES)
            pl.when(k == ks)(lambda wlen=wlen: pltpu.make_async_copy(
                comp_hbm.at[pl.ds(r, wlen)], win.at[pl.ds(0, wlen)],
                sem).start())

    def wait_in(t, win, sem):
        i = _row_id(wid, t)
        _, _, _, k = _win_params(i)
        for ks in range(NUM_CLS):
            wlen = WIN - ks * (CLASS_STEP * LANES)
            pl.when(k == ks)(lambda wlen=wlen: pltpu.make_async_copy(
                comp_hbm.at[pl.ds(0, wlen)], win.at[pl.ds(0, wlen)],
                sem).wait())

    def wait_out(row, sem):
        pltpu.make_async_copy(row, out_hbm.at[0], sem).wait()

    def build(rowbuf, win, i):
        d, _, cb, _ = _win_params(i)

        @pl.when(i < N - 1)
        def _():
            c16 = cb * LANES
            idx = jnp.maximum(lane + (d + c16), 0)
            v = plsc.load_gather(win, [idx])
            keep = (lane + c16) >= i + 1
            rowbuf[pl.ds(c16, LANES)] = jnp.where(keep, v, 0.0)

        @plsc.parallel_loop(cb + 1, CHUNKS, unroll=UNROLL)
        def _(c):
            c16 = c * LANES
            v = plsc.load_gather(win, [lane + (d + c16)])
            rowbuf[pl.ds(c16, LANES)] = v

    for t in range(DEPTH - 1):
        start_in(t, wins[t], sin[t])

    def group(g, _):
        t0 = DEPTH * g
        for b in range(DEPTH):
            t = t0 + b
            i = _row_id(wid, t)
            wait_in(t, wins[b], sin[b])
            pn = (b + DEPTH - 1) % DEPTH
            pl.when(t + DEPTH - 1 < ROWS_PER_TILE)(
                lambda t=t, pn=pn: start_in(t + DEPTH - 1, wins[pn], sin[pn]))
            pl.when(g > 0)(lambda b=b: wait_out(rows[b], sout[b]))
            build(rows[b], wins[b], i)
            pltpu.make_async_copy(rows[b], out_hbm.at[i], sout[b]).start()
        return 0
    lax.fori_loop(0, ROWS_PER_TILE // DEPTH, group, 0)

    for b in range(DEPTH):
        wait_out(rows[b], sout[b])


@jax.jit
def kernel(compressed_matrix):
    run = pl.kernel(
        _body,
        out_type=jax.ShapeDtypeStruct((N, N), jnp.float32),
        mesh=plsc.VectorSubcoreMesh(core_axis_name="c", subcore_axis_name="s"),
        scratch_types=(
            [pltpu.VMEM((WIN,), jnp.float32) for _ in range(DEPTH)]
            + [pltpu.VMEM((N,), jnp.float32) for _ in range(DEPTH)]
            + [pltpu.SemaphoreType.DMA for _ in range(2 * DEPTH)]
        ),
        compiler_params=pltpu.CompilerParams(needs_layout_passes=False),
    )
    return run(compressed_matrix)
